# baseline (speedup 1.0000x reference)
import dataclasses

import jax
import jax.numpy as jnp
from jax import lax
from jax.experimental import pallas as pl
from jax.experimental.pallas import tpu as pltpu
from jax.experimental.pallas import tpu_sc as plsc

N_NODES = 10000
N_EDGES = 320000
F = 128

E_TILE = 2000
N_TILE = 1000
N_HALF = 2

SC_TILES = 16
SC_WORKERS = 2 * SC_TILES
N_PAD = 10240
ROWS_PER_TILE = N_PAD // SC_TILES


FH = F // 2


def _relu(v):
    return jnp.maximum(v, 0.0)


def _mm(a, b):
    return jnp.dot(a.astype(jnp.bfloat16), b, preferred_element_type=jnp.float32)


def _pack_bf16(p):
    u = lax.bitcast_convert_type(
        p[:, :FH].astype(jnp.bfloat16), jnp.uint16).astype(jnp.uint32)
    v = lax.bitcast_convert_type(
        p[:, FH:].astype(jnp.bfloat16), jnp.uint16).astype(jnp.uint32)
    return lax.bitcast_convert_type(u | (v << 16), jnp.float32)


def _unpack_bf16(gp):
    w = lax.bitcast_convert_type(gp, jnp.uint32)
    lo = lax.bitcast_convert_type((w & 0xFFFF).astype(jnp.uint16), jnp.bfloat16)
    hi = lax.bitcast_convert_type((w >> 16).astype(jnp.uint16), jnp.bfloat16)
    return jnp.concatenate(
        [lo.astype(jnp.float32), hi.astype(jnp.float32)], axis=-1)


def _rowcall(body, n_rows, tile, data, weights, out_dims):
    grid = (n_rows // tile,)
    in_specs = (
        [pl.BlockSpec((tile, a.shape[1]), lambda i: (i, 0)) for a in data]
        + [pl.BlockSpec(w.shape, lambda i: (0,) * w.ndim) for w in weights]
    )
    out_specs = [pl.BlockSpec((tile, d), lambda i: (i, 0)) for d in out_dims]
    out_shape = [jax.ShapeDtypeStruct((n_rows, d), jnp.float32) for d in out_dims]
    out = pl.pallas_call(
        body, grid=grid, in_specs=in_specs, out_specs=out_specs, out_shape=out_shape
    )(*data, *weights)
    return out


def _node_enc_body(na, W0, b0, W1, b1, Ws, Wd, x_o, ps_o, pd_o):
    h = _relu(_mm(na[...], W0[...]) + b0[...])
    x = _mm(h, W1[...]) + b1[...]
    x_o[...] = x
    ps_o[...] = _mm(x, Ws[...])
    pd_o[...] = _mm(x, Wd[...])


def _edge1_body(ea, g, We0, be0, We1, be1, Wm0e, bm0, Wm1, bm1,
                Wu0, bu0, Wu1, bu1, m_o, e_o):
    h0 = _relu(_mm(ea[...], We0[...]) + be0[...])
    e = _mm(h0, We1[...]) + be1[...]
    h = _relu(_mm(e, Wm0e[...]) + _unpack_bf16(g[...]) + bm0[...])
    m = _mm(h, Wm1[...]) + bm1[...]
    hu = _relu(_mm(m, Wu0[...]) + bu0[...])
    m_o[...] = m
    e_o[...] = _mm(hu, Wu1[...]) + bu1[...] + e


def _edge_mp_body(e_in, g, Wm0e, bm0, Wm1, bm1, Wu0, bu0, Wu1, bu1, m_o, e_o):
    e = e_in[...]
    h = _relu(_mm(e, Wm0e[...]) + _unpack_bf16(g[...]) + bm0[...])
    m = _mm(h, Wm1[...]) + bm1[...]
    hu = _relu(_mm(m, Wu0[...]) + bu0[...])
    m_o[...] = m
    e_o[...] = _mm(hu, Wu1[...]) + bu1[...] + e


def _node_mp_body(x_in, a0, a1, Wnx, Wna, bn0, Wn1, bn1, Ws, Wd,
                  x_o, ps_o, pd_o):
    x = x_in[...]
    aggr = a0[...] + a1[...]
    h = _relu(_mm(x, Wnx[...]) + _mm(aggr, Wna[...]) + bn0[...])
    x2 = _mm(h, Wn1[...]) + bn1[...] + x
    x_o[...] = x2
    ps_o[...] = _mm(x2, Ws[...])
    pd_o[...] = _mm(x2, Wd[...])


def _node_mp_dec_body(x_in, a0, a1, Wnx, Wna, bn0, Wn1, bn1,
                      Wc0, bc0, Wc1, bc1, Ws, Wd, no_o, ps_o, pd_o):
    x = x_in[...]
    aggr = a0[...] + a1[...]
    h = _relu(_mm(x, Wnx[...]) + _mm(aggr, Wna[...]) + bn0[...])
    x2 = _mm(h, Wn1[...]) + bn1[...] + x
    hd = _relu(_mm(x2, Wc0[...]) + bc0[...])
    no_o[...] = _mm(hd, Wc1[...]) + bc1[...]
    ps_o[...] = _mm(x2, Ws[...])
    pd_o[...] = _mm(x2, Wd[...])


def _edge_dec_body(e_in, g, Wd0e, bd0, Wd1, bd1, o):
    h = _relu(_mm(e_in[...], Wd0e[...]) + _unpack_bf16(g[...]) + bd0[...])
    o[...] = _mm(h, Wd1[...]) + bd1[...]


def _sc_mesh():
    return plsc.VectorSubcoreMesh(core_axis_name="c", subcore_axis_name="s")


def _sc_params():
    cp = pltpu.CompilerParams()
    if "needs_layout_passes" in pltpu.CompilerParams.__dataclass_fields__:
        cp = dataclasses.replace(cp, needs_layout_passes=False)
    return cp


def _chunking(n_edges):
    epw = n_edges // SC_WORKERS
    ch = 80 if epw % 80 == 0 else 40
    n_chunks = epw // ch
    assert epw % ch == 0 and n_chunks % 2 == 1
    return epw, ch, n_chunks


def _sc_gather_add(ps, pd, src, dst):
    n_edges = src.shape[0]
    epw, ch, n_chunks = _chunking(n_edges)

    def body(ps_hbm, pd_hbm, src_hbm, dst_hbm, g_hbm,
             ixs, ixd, bs0, bd0, bo0, bs1, bd1, bo1, sg0, sg1, ss0, ss1):
        wid = lax.axis_index("c") * SC_TILES + lax.axis_index("s")
        tb = wid * epw
        pltpu.sync_copy(src_hbm.at[pl.ds(tb, epw)], ixs)
        pltpu.sync_copy(dst_hbm.at[pl.ds(tb, epw)], ixd)

        def start(kk, bs, bd, sg):
            off = kk * ch
            c1 = pltpu.async_copy(ps_hbm.at[ixs.at[pl.ds(off, ch)]], bs, sg)
            c2 = pltpu.async_copy(pd_hbm.at[ixd.at[pl.ds(off, ch)]], bd, sg)
            return c1, c2

        def finish(kk, cps, bs, bd, bo, ss, guard):
            base = tb + kk * ch

            @pl.when(guard)
            def _wait_prev_store():
                pltpu.make_async_copy(bo, g_hbm.at[pl.ds(base, ch)], ss).wait()

            cps[0].wait()
            cps[1].wait()

            @pl.loop(0, ch, unroll=4)
            def _row(r):
                for cc in range(0, FH, 16):
                    a = bs[r, pl.ds(cc, 16)] + bd[r, pl.ds(cc, 16)]
                    b = bs[r, pl.ds(cc + FH, 16)] + bd[r, pl.ds(cc + FH, 16)]
                    packed = plsc.pack(a, b, format=plsc.PackFormat.INTERLEAVED)
                    bo[r, pl.ds(cc, 16)] = plsc.bitcast(packed, jnp.float32)

            pltpu.async_copy(bo, g_hbm.at[pl.ds(base, ch)], ss)

        @pl.loop(0, n_chunks - 1, step=2)
        def _pair(k):
            ca = start(k, bs0, bd0, sg0)
            cb = start(k + 1, bs1, bd1, sg1)
            finish(k, ca, bs0, bd0, bo0, ss0, k > 0)
            finish(k + 1, cb, bs1, bd1, bo1, ss1, k > 0)

        klast = n_chunks - 1
        ca = start(klast, bs0, bd0, sg0)
        finish(klast, ca, bs0, bd0, bo0, ss0, True)
        pltpu.make_async_copy(
            bo0, g_hbm.at[pl.ds(tb + klast * ch, ch)], ss0).wait()
        pltpu.make_async_copy(
            bo1, g_hbm.at[pl.ds(tb + (klast - 1) * ch, ch)], ss1).wait()

    f = pl.kernel(
        body,
        out_type=jax.ShapeDtypeStruct((n_edges, FH), jnp.float32),
        mesh=_sc_mesh(),
        compiler_params=_sc_params(),
        scratch_types=[
            pltpu.VMEM((epw,), jnp.int32),
            pltpu.VMEM((epw,), jnp.int32),
            pltpu.VMEM((ch, F), jnp.float32),
            pltpu.VMEM((ch, F), jnp.float32),
            pltpu.VMEM((ch, FH), jnp.float32),
            pltpu.VMEM((ch, F), jnp.float32),
            pltpu.VMEM((ch, F), jnp.float32),
            pltpu.VMEM((ch, FH), jnp.float32),
            pltpu.SemaphoreType.DMA,
            pltpu.SemaphoreType.DMA,
            pltpu.SemaphoreType.DMA,
            pltpu.SemaphoreType.DMA,
        ],
    )
    return f(ps, pd, src, dst)


def _sc_segment_sum(ms, dsts):
    n_half = ms[0].shape[0]
    epw = n_half // SC_TILES
    ch = 80
    n_chunks = epw // ch
    assert epw % ch == 0 and n_chunks % 2 == 1

    def body(m0_hbm, m1_hbm, d0_hbm, d1_hbm, out_hbm,
             mb0, mb1, ix0, ix1, shared, sm0, sm1):
        c = lax.axis_index("c")
        s = lax.axis_index("s")

        @pl.loop(0, ch)
        def _zrow(r):
            @pl.loop(0, F, step=16)
            def _zcol(cc):
                mb0[r, pl.ds(cc, 16)] = jnp.zeros((16,), jnp.float32)

        @pl.loop(0, ROWS_PER_TILE, step=ch)
        def _zcopy(j):
            pltpu.sync_copy(mb0, shared.at[pl.ds(s * ROWS_PER_TILE + j, ch)])

        plsc.subcore_barrier()

        tb = s * epw

        def run(m_hbm, d_hbm):
            @pl.loop(0, n_chunks - 1, step=2)
            def _pair(k):
                ca = pltpu.async_copy(
                    m_hbm.at[pl.ds(tb + k * ch, ch)], mb0, sm0)
                cb = pltpu.async_copy(
                    m_hbm.at[pl.ds(tb + (k + 1) * ch, ch)], mb1, sm1)
                pltpu.sync_copy(d_hbm.at[pl.ds(tb + k * ch, ch)], ix0)
                pltpu.sync_copy(d_hbm.at[pl.ds(tb + (k + 1) * ch, ch)], ix1)
                ca.wait()
                pltpu.sync_copy(mb0, shared.at[ix0], add=True)
                cb.wait()
                pltpu.sync_copy(mb1, shared.at[ix1], add=True)

            klast = n_chunks - 1
            cl = pltpu.async_copy(
                m_hbm.at[pl.ds(tb + klast * ch, ch)], mb0, sm0)
            pltpu.sync_copy(d_hbm.at[pl.ds(tb + klast * ch, ch)], ix0)
            cl.wait()
            pltpu.sync_copy(mb0, shared.at[ix0], add=True)

        @pl.when(c == 0)
        def _half0():
            run(m0_hbm, d0_hbm)

        @pl.when(c == 1)
        def _half1():
            run(m1_hbm, d1_hbm)

        plsc.subcore_barrier()
        out_base = c * N_PAD + s * ROWS_PER_TILE
        pltpu.sync_copy(
            shared.at[pl.ds(s * ROWS_PER_TILE, ROWS_PER_TILE)],
            out_hbm.at[pl.ds(out_base, ROWS_PER_TILE)],
        )

    f = pl.kernel(
        body,
        out_type=jax.ShapeDtypeStruct((2 * N_PAD, F), jnp.float32),
        mesh=_sc_mesh(),
        scratch_types=[
            pltpu.VMEM((ch, F), jnp.float32),
            pltpu.VMEM((ch, F), jnp.float32),
            pltpu.VMEM((ch,), jnp.int32),
            pltpu.VMEM((ch,), jnp.int32),
            pltpu.VMEM_SHARED((N_PAD, F), jnp.float32),
            pltpu.SemaphoreType.DMA,
            pltpu.SemaphoreType.DMA,
        ],
    )
    return f(ms[0], ms[1], dsts[0], dsts[1])


def kernel(node_attr, edge_index, edge_attr, params):
    eh = N_EDGES // N_HALF
    srcs = [edge_index[0, i * eh:(i + 1) * eh] for i in range(N_HALF)]
    dsts = [edge_index[1, i * eh:(i + 1) * eh] for i in range(N_HALF)]
    eas = [edge_attr[i * eh:(i + 1) * eh] for i in range(N_HALF)]

    (Wne0, bne0), (Wne1, bne1) = params["node_enc"]
    (Wee0, bee0), (Wee1, bee1) = params["edge_enc"]
    (Wnd0, bnd0), (Wnd1, bnd1) = params["node_dec"]
    (Wd0, bd0), (Wd1, bd1) = params["edge_dec"]
    mp = params["mp"]

    def msg_split(lp):
        (Wm0, bm0), (Wm1, bm1) = lp["msg"]
        return Wm0[:F], Wm0[F:2 * F], Wm0[2 * F:], bm0, Wm1, bm1

    def node_split(lp):
        (Wn0, bn0), (Wn1, bn1) = lp["node"]
        return Wn0[:F], Wn0[F:], bn0, Wn1, bn1

    r = lambda b: b.reshape(1, -1)
    w = lambda W: W.astype(jnp.bfloat16)

    Wm0e_1, Wm0s_1, Wm0d_1, bm0_1, Wm1_1, bm1_1 = msg_split(mp[0])
    Wm0e_2, Wm0s_2, Wm0d_2, bm0_2, Wm1_2, bm1_2 = msg_split(mp[1])
    (Wu0_1, bu0_1), (Wu1_1, bu1_1) = mp[0]["edge"]
    (Wu0_2, bu0_2), (Wu1_2, bu1_2) = mp[1]["edge"]
    Wnx_1, Wna_1, bn0_1, Wn1_1, bn1_1 = node_split(mp[0])
    Wnx_2, Wna_2, bn0_2, Wn1_2, bn1_2 = node_split(mp[1])
    Wd0e, Wd0s, Wd0d = Wd0[:F], Wd0[F:2 * F], Wd0[2 * F:]

    x0, p1s, p1d = _rowcall(
        _node_enc_body, N_NODES, N_TILE, [node_attr],
        [w(Wne0), r(bne0), w(Wne1), r(bne1), w(Wm0s_1), w(Wm0d_1)], [F, F, F])

    g1 = [_sc_gather_add(p1s, p1d, srcs[i], dsts[i]) for i in range(N_HALF)]
    me1 = [_rowcall(
        _edge1_body, eh, E_TILE, [eas[i], g1[i]],
        [w(Wee0), r(bee0), w(Wee1), r(bee1), w(Wm0e_1), r(bm0_1), w(Wm1_1),
         r(bm1_1), w(Wu0_1), r(bu0_1), w(Wu1_1), r(bu1_1)], [F, F]) for i in range(N_HALF)]
    parts1 = _sc_segment_sum([me1[0][0], me1[1][0]], dsts)
    x1, p2s, p2d = _rowcall(
        _node_mp_body, N_NODES, N_TILE,
        [x0, parts1[:N_NODES], parts1[N_PAD:N_PAD + N_NODES]],
        [w(Wnx_1), w(Wna_1), r(bn0_1), w(Wn1_1), r(bn1_1), w(Wm0s_2), w(Wm0d_2)], [F, F, F])

    g2 = [_sc_gather_add(p2s, p2d, srcs[i], dsts[i]) for i in range(N_HALF)]
    me2 = [_rowcall(
        _edge_mp_body, eh, E_TILE, [me1[i][1], g2[i]],
        [w(Wm0e_2), r(bm0_2), w(Wm1_2), r(bm1_2),
         w(Wu0_2), r(bu0_2), w(Wu1_2), r(bu1_2)], [F, F]) for i in range(N_HALF)]
    parts2 = _sc_segment_sum([me2[0][0], me2[1][0]], dsts)
    node_out, pds, pdd = _rowcall(
        _node_mp_dec_body, N_NODES, N_TILE,
        [x1, parts2[:N_NODES], parts2[N_PAD:N_PAD + N_NODES]],
        [w(Wnx_2), w(Wna_2), r(bn0_2), w(Wn1_2), r(bn1_2),
         w(Wnd0), r(bnd0), w(Wnd1), r(bnd1), w(Wd0s), w(Wd0d)], [F, F, F])

    g3 = [_sc_gather_add(pds, pdd, srcs[i], dsts[i]) for i in range(N_HALF)]
    eo = [_rowcall(
        _edge_dec_body, eh, E_TILE, [me2[i][1], g3[i]],
        [w(Wd0e), r(bd0), w(Wd1), r(bd1)], [16])[0] for i in range(N_HALF)]
    edge_out = jnp.concatenate(eo, axis=0)

    return (node_out, edge_out)


# speedup vs baseline: 1.0358x; 1.0358x over previous
import dataclasses

import jax
import jax.numpy as jnp
from jax import lax
from jax.experimental import pallas as pl
from jax.experimental.pallas import tpu as pltpu
from jax.experimental.pallas import tpu_sc as plsc

N_NODES = 10000
N_EDGES = 320000
F = 128

E_TILE = 2000
N_TILE = 1000
N_HALF = 2

SC_TILES = 16
SC_WORKERS = 2 * SC_TILES
N_PAD = 10240
ROWS_PER_TILE = N_PAD // SC_TILES


FH = F // 2


def _relu(v):
    return jnp.maximum(v, 0.0)


def _mm(a, b):
    return jnp.dot(a.astype(jnp.bfloat16), b, preferred_element_type=jnp.float32)


def _pack_bf16(p):
    u = lax.bitcast_convert_type(
        p[:, :FH].astype(jnp.bfloat16), jnp.uint16).astype(jnp.uint32)
    v = lax.bitcast_convert_type(
        p[:, FH:].astype(jnp.bfloat16), jnp.uint16).astype(jnp.uint32)
    return lax.bitcast_convert_type(u | (v << 16), jnp.float32)


def _unpack_bf16(gp):
    w = lax.bitcast_convert_type(gp, jnp.uint32)
    lo = lax.bitcast_convert_type((w & 0xFFFF).astype(jnp.uint16), jnp.bfloat16)
    hi = lax.bitcast_convert_type((w >> 16).astype(jnp.uint16), jnp.bfloat16)
    return jnp.concatenate(
        [lo.astype(jnp.float32), hi.astype(jnp.float32)], axis=-1)


def _rowcall(body, n_rows, tile, data, weights, out_dims):
    grid = (n_rows // tile,)
    in_specs = (
        [pl.BlockSpec((tile, a.shape[1]), lambda i: (i, 0)) for a in data]
        + [pl.BlockSpec(w.shape, lambda i: (0,) * w.ndim) for w in weights]
    )
    out_specs = [pl.BlockSpec((tile, d), lambda i: (i, 0)) for d in out_dims]
    out_shape = [jax.ShapeDtypeStruct((n_rows, d), jnp.float32) for d in out_dims]
    out = pl.pallas_call(
        body, grid=grid, in_specs=in_specs, out_specs=out_specs, out_shape=out_shape
    )(*data, *weights)
    return out


def _node_enc_body(na, W0, b0, W1, b1, Ws, Wd, x_o, ps_o, pd_o):
    h = _relu(_mm(na[...], W0[...]) + b0[...])
    x = _mm(h, W1[...]) + b1[...]
    x_o[...] = x
    ps_o[...] = _mm(x, Ws[...])
    pd_o[...] = _mm(x, Wd[...])


def _edge1_body(ea, g, We0, be0, We1, be1, Wm0e, bm0, Wm1, bm1,
                Wu0, bu0, Wu1, bu1, m_o, e_o):
    h0 = _relu(_mm(ea[...], We0[...]) + be0[...])
    e = _mm(h0, We1[...]) + be1[...]
    h = _relu(_mm(e, Wm0e[...]) + _unpack_bf16(g[...]) + bm0[...])
    m = _mm(h, Wm1[...]) + bm1[...]
    hu = _relu(_mm(m, Wu0[...]) + bu0[...])
    m_o[...] = m
    e_o[...] = _mm(hu, Wu1[...]) + bu1[...] + e


def _edge_mp_body(e_in, g, Wm0e, bm0, Wm1, bm1, Wu0, bu0, Wu1, bu1, m_o, e_o):
    e = e_in[...]
    h = _relu(_mm(e, Wm0e[...]) + _unpack_bf16(g[...]) + bm0[...])
    m = _mm(h, Wm1[...]) + bm1[...]
    hu = _relu(_mm(m, Wu0[...]) + bu0[...])
    m_o[...] = m
    e_o[...] = _mm(hu, Wu1[...]) + bu1[...] + e


def _node_mp_body(x_in, a0, a1, a2, a3, Wnx, Wna, bn0, Wn1, bn1, Ws, Wd,
                  x_o, ps_o, pd_o):
    x = x_in[...]
    aggr = (a0[...] + a1[...]) + (a2[...] + a3[...])
    h = _relu(_mm(x, Wnx[...]) + _mm(aggr, Wna[...]) + bn0[...])
    x2 = _mm(h, Wn1[...]) + bn1[...] + x
    x_o[...] = x2
    ps_o[...] = _mm(x2, Ws[...])
    pd_o[...] = _mm(x2, Wd[...])


def _node_mp_dec_body(x_in, a0, a1, a2, a3, Wnx, Wna, bn0, Wn1, bn1,
                      Wc0, bc0, Wc1, bc1, Ws, Wd, no_o, ps_o, pd_o):
    x = x_in[...]
    aggr = (a0[...] + a1[...]) + (a2[...] + a3[...])
    h = _relu(_mm(x, Wnx[...]) + _mm(aggr, Wna[...]) + bn0[...])
    x2 = _mm(h, Wn1[...]) + bn1[...] + x
    hd = _relu(_mm(x2, Wc0[...]) + bc0[...])
    no_o[...] = _mm(hd, Wc1[...]) + bc1[...]
    ps_o[...] = _mm(x2, Ws[...])
    pd_o[...] = _mm(x2, Wd[...])


def _edge_dec_body(e_in, g, Wd0e, bd0, Wd1, bd1, o):
    h = _relu(_mm(e_in[...], Wd0e[...]) + _unpack_bf16(g[...]) + bd0[...])
    o[...] = _mm(h, Wd1[...]) + bd1[...]


def _sc_mesh():
    return plsc.VectorSubcoreMesh(core_axis_name="c", subcore_axis_name="s")


def _sc_params():
    cp = pltpu.CompilerParams()
    if "needs_layout_passes" in pltpu.CompilerParams.__dataclass_fields__:
        cp = dataclasses.replace(cp, needs_layout_passes=False)
    return cp


def _chunking(n_edges):
    epw = n_edges // SC_WORKERS
    ch = 80 if epw % 80 == 0 else 40
    n_chunks = epw // ch
    assert epw % ch == 0 and n_chunks % 2 == 1
    return epw, ch, n_chunks


def _sc_gather_add(ps, pd, src, dst):
    n_edges = src.shape[0]
    epw, ch, n_chunks = _chunking(n_edges)

    def body(ps_hbm, pd_hbm, src_hbm, dst_hbm, g_hbm,
             ixs, ixd, bs0, bd0, bo0, bs1, bd1, bo1, sg0, sg1, ss0, ss1):
        wid = lax.axis_index("c") * SC_TILES + lax.axis_index("s")
        tb = wid * epw
        pltpu.sync_copy(src_hbm.at[pl.ds(tb, epw)], ixs)
        pltpu.sync_copy(dst_hbm.at[pl.ds(tb, epw)], ixd)

        def start(kk, bs, bd, sg):
            off = kk * ch
            c1 = pltpu.async_copy(ps_hbm.at[ixs.at[pl.ds(off, ch)]], bs, sg)
            c2 = pltpu.async_copy(pd_hbm.at[ixd.at[pl.ds(off, ch)]], bd, sg)
            return c1, c2

        def finish(kk, cps, bs, bd, bo, ss, guard):
            base = tb + kk * ch

            @pl.when(guard)
            def _wait_prev_store():
                pltpu.make_async_copy(bo, g_hbm.at[pl.ds(base, ch)], ss).wait()

            cps[0].wait()
            cps[1].wait()

            @pl.loop(0, ch, unroll=4)
            def _row(r):
                for cc in range(0, FH, 16):
                    a = bs[r, pl.ds(cc, 16)] + bd[r, pl.ds(cc, 16)]
                    b = bs[r, pl.ds(cc + FH, 16)] + bd[r, pl.ds(cc + FH, 16)]
                    packed = plsc.pack(a, b, format=plsc.PackFormat.INTERLEAVED)
                    bo[r, pl.ds(cc, 16)] = plsc.bitcast(packed, jnp.float32)

            pltpu.async_copy(bo, g_hbm.at[pl.ds(base, ch)], ss)

        @pl.loop(0, n_chunks - 1, step=2)
        def _pair(k):
            ca = start(k, bs0, bd0, sg0)
            cb = start(k + 1, bs1, bd1, sg1)
            finish(k, ca, bs0, bd0, bo0, ss0, k > 0)
            finish(k + 1, cb, bs1, bd1, bo1, ss1, k > 0)

        klast = n_chunks - 1
        ca = start(klast, bs0, bd0, sg0)
        finish(klast, ca, bs0, bd0, bo0, ss0, True)
        pltpu.make_async_copy(
            bo0, g_hbm.at[pl.ds(tb + klast * ch, ch)], ss0).wait()
        pltpu.make_async_copy(
            bo1, g_hbm.at[pl.ds(tb + (klast - 1) * ch, ch)], ss1).wait()

    f = pl.kernel(
        body,
        out_type=jax.ShapeDtypeStruct((n_edges, FH), jnp.float32),
        mesh=_sc_mesh(),
        compiler_params=_sc_params(),
        scratch_types=[
            pltpu.VMEM((epw,), jnp.int32),
            pltpu.VMEM((epw,), jnp.int32),
            pltpu.VMEM((ch, F), jnp.float32),
            pltpu.VMEM((ch, F), jnp.float32),
            pltpu.VMEM((ch, FH), jnp.float32),
            pltpu.VMEM((ch, F), jnp.float32),
            pltpu.VMEM((ch, F), jnp.float32),
            pltpu.VMEM((ch, FH), jnp.float32),
            pltpu.SemaphoreType.DMA,
            pltpu.SemaphoreType.DMA,
            pltpu.SemaphoreType.DMA,
            pltpu.SemaphoreType.DMA,
        ],
    )
    return f(ps, pd, src, dst)


def _sc_segment_sum(m, dst):
    n_edges = m.shape[0]
    epw, ch, n_chunks = _chunking(n_edges)

    def body(m_hbm, dst_hbm, out_hbm, mb0, mb1, ix0, ix1, shared, sm0, sm1):
        c = lax.axis_index("c")
        s = lax.axis_index("s")
        wid = c * SC_TILES + s

        @pl.loop(0, ch)
        def _zrow(r):
            @pl.loop(0, F, step=16)
            def _zcol(cc):
                mb0[r, pl.ds(cc, 16)] = jnp.zeros((16,), jnp.float32)

        @pl.loop(0, ROWS_PER_TILE, step=ch)
        def _zcopy(j):
            pltpu.sync_copy(mb0, shared.at[pl.ds(s * ROWS_PER_TILE + j, ch)])

        plsc.subcore_barrier()

        tb = wid * epw

        @pl.loop(0, n_chunks - 1, step=2)
        def _pair(k):
            ca = pltpu.async_copy(m_hbm.at[pl.ds(tb + k * ch, ch)], mb0, sm0)
            cb = pltpu.async_copy(
                m_hbm.at[pl.ds(tb + (k + 1) * ch, ch)], mb1, sm1)
            pltpu.sync_copy(dst_hbm.at[pl.ds(tb + k * ch, ch)], ix0)
            pltpu.sync_copy(dst_hbm.at[pl.ds(tb + (k + 1) * ch, ch)], ix1)
            ca.wait()
            pltpu.sync_copy(mb0, shared.at[ix0], add=True)
            cb.wait()
            pltpu.sync_copy(mb1, shared.at[ix1], add=True)

        klast = n_chunks - 1
        cl = pltpu.async_copy(m_hbm.at[pl.ds(tb + klast * ch, ch)], mb0, sm0)
        pltpu.sync_copy(dst_hbm.at[pl.ds(tb + klast * ch, ch)], ix0)
        cl.wait()
        pltpu.sync_copy(mb0, shared.at[ix0], add=True)

        plsc.subcore_barrier()
        out_base = c * N_PAD + s * ROWS_PER_TILE
        pltpu.sync_copy(
            shared.at[pl.ds(s * ROWS_PER_TILE, ROWS_PER_TILE)],
            out_hbm.at[pl.ds(out_base, ROWS_PER_TILE)],
        )

    f = pl.kernel(
        body,
        out_type=jax.ShapeDtypeStruct((2 * N_PAD, F), jnp.float32),
        mesh=_sc_mesh(),
        scratch_types=[
            pltpu.VMEM((ch, F), jnp.float32),
            pltpu.VMEM((ch, F), jnp.float32),
            pltpu.VMEM((ch,), jnp.int32),
            pltpu.VMEM((ch,), jnp.int32),
            pltpu.VMEM_SHARED((N_PAD, F), jnp.float32),
            pltpu.SemaphoreType.DMA,
            pltpu.SemaphoreType.DMA,
        ],
    )
    return f(m, dst)


def kernel(node_attr, edge_index, edge_attr, params):
    eh = N_EDGES // N_HALF
    srcs = [edge_index[0, i * eh:(i + 1) * eh] for i in range(N_HALF)]
    dsts = [edge_index[1, i * eh:(i + 1) * eh] for i in range(N_HALF)]
    eas = [edge_attr[i * eh:(i + 1) * eh] for i in range(N_HALF)]

    (Wne0, bne0), (Wne1, bne1) = params["node_enc"]
    (Wee0, bee0), (Wee1, bee1) = params["edge_enc"]
    (Wnd0, bnd0), (Wnd1, bnd1) = params["node_dec"]
    (Wd0, bd0), (Wd1, bd1) = params["edge_dec"]
    mp = params["mp"]

    def msg_split(lp):
        (Wm0, bm0), (Wm1, bm1) = lp["msg"]
        return Wm0[:F], Wm0[F:2 * F], Wm0[2 * F:], bm0, Wm1, bm1

    def node_split(lp):
        (Wn0, bn0), (Wn1, bn1) = lp["node"]
        return Wn0[:F], Wn0[F:], bn0, Wn1, bn1

    r = lambda b: b.reshape(1, -1)
    w = lambda W: W.astype(jnp.bfloat16)

    Wm0e_1, Wm0s_1, Wm0d_1, bm0_1, Wm1_1, bm1_1 = msg_split(mp[0])
    Wm0e_2, Wm0s_2, Wm0d_2, bm0_2, Wm1_2, bm1_2 = msg_split(mp[1])
    (Wu0_1, bu0_1), (Wu1_1, bu1_1) = mp[0]["edge"]
    (Wu0_2, bu0_2), (Wu1_2, bu1_2) = mp[1]["edge"]
    Wnx_1, Wna_1, bn0_1, Wn1_1, bn1_1 = node_split(mp[0])
    Wnx_2, Wna_2, bn0_2, Wn1_2, bn1_2 = node_split(mp[1])
    Wd0e, Wd0s, Wd0d = Wd0[:F], Wd0[F:2 * F], Wd0[2 * F:]

    x0, p1s, p1d = _rowcall(
        _node_enc_body, N_NODES, N_TILE, [node_attr],
        [w(Wne0), r(bne0), w(Wne1), r(bne1), w(Wm0s_1), w(Wm0d_1)], [F, F, F])

    g1 = [_sc_gather_add(p1s, p1d, srcs[i], dsts[i]) for i in range(N_HALF)]
    me1 = [_rowcall(
        _edge1_body, eh, E_TILE, [eas[i], g1[i]],
        [w(Wee0), r(bee0), w(Wee1), r(bee1), w(Wm0e_1), r(bm0_1), w(Wm1_1),
         r(bm1_1), w(Wu0_1), r(bu0_1), w(Wu1_1), r(bu1_1)], [F, F]) for i in range(N_HALF)]
    parts1 = [_sc_segment_sum(me1[i][0], dsts[i]) for i in range(N_HALF)]
    x1, p2s, p2d = _rowcall(
        _node_mp_body, N_NODES, N_TILE,
        [x0,
         parts1[0][:N_NODES], parts1[0][N_PAD:N_PAD + N_NODES],
         parts1[1][:N_NODES], parts1[1][N_PAD:N_PAD + N_NODES]],
        [w(Wnx_1), w(Wna_1), r(bn0_1), w(Wn1_1), r(bn1_1), w(Wm0s_2), w(Wm0d_2)], [F, F, F])

    g2 = [_sc_gather_add(p2s, p2d, srcs[i], dsts[i]) for i in range(N_HALF)]
    me2 = [_rowcall(
        _edge_mp_body, eh, E_TILE, [me1[i][1], g2[i]],
        [w(Wm0e_2), r(bm0_2), w(Wm1_2), r(bm1_2),
         w(Wu0_2), r(bu0_2), w(Wu1_2), r(bu1_2)], [F, F]) for i in range(N_HALF)]
    parts2 = [_sc_segment_sum(me2[i][0], dsts[i]) for i in range(N_HALF)]
    node_out, pds, pdd = _rowcall(
        _node_mp_dec_body, N_NODES, N_TILE,
        [x1,
         parts2[0][:N_NODES], parts2[0][N_PAD:N_PAD + N_NODES],
         parts2[1][:N_NODES], parts2[1][N_PAD:N_PAD + N_NODES]],
        [w(Wnx_2), w(Wna_2), r(bn0_2), w(Wn1_2), r(bn1_2),
         w(Wnd0), r(bnd0), w(Wnd1), r(bnd1), w(Wd0s), w(Wd0d)], [F, F, F])

    g3 = [_sc_gather_add(pds, pdd, srcs[i], dsts[i]) for i in range(N_HALF)]
    eo = [_rowcall(
        _edge_dec_body, eh, E_TILE, [me2[i][1], g3[i]],
        [w(Wd0e), r(bd0), w(Wd1), r(bd1)], [16])[0] for i in range(N_HALF)]
    edge_out = jnp.concatenate(eo, axis=0)

    return (node_out, edge_out)


# speedup vs baseline: 1.2352x; 1.1926x over previous
import dataclasses

import jax
import jax.numpy as jnp
from jax import lax
from jax.experimental import pallas as pl
from jax.experimental.pallas import tpu as pltpu
from jax.experimental.pallas import tpu_sc as plsc

N_NODES = 10000
N_EDGES = 320000
F = 128

E_TILE = 2000
N_TILE = 1000
N_HALF = 2

SC_TILES = 16
SC_WORKERS = 2 * SC_TILES
N_PAD = 10240
ROWS_PER_TILE = N_PAD // SC_TILES


FH = F // 2


def _relu(v):
    return jnp.maximum(v, 0.0)


def _mm(a, b):
    return jnp.dot(a.astype(jnp.bfloat16), b, preferred_element_type=jnp.float32)


def _pack_bf16(p):
    u = lax.bitcast_convert_type(
        p[:, :FH].astype(jnp.bfloat16), jnp.uint16).astype(jnp.uint32)
    v = lax.bitcast_convert_type(
        p[:, FH:].astype(jnp.bfloat16), jnp.uint16).astype(jnp.uint32)
    return lax.bitcast_convert_type(u | (v << 16), jnp.float32)


def _unpack_bf16(gp):
    w = lax.bitcast_convert_type(gp, jnp.uint32)
    lo = lax.bitcast_convert_type((w & 0xFFFF).astype(jnp.uint16), jnp.bfloat16)
    hi = lax.bitcast_convert_type((w >> 16).astype(jnp.uint16), jnp.bfloat16)
    return jnp.concatenate(
        [lo.astype(jnp.float32), hi.astype(jnp.float32)], axis=-1)


def _rowcall(body, n_rows, tile, data, weights, out_dims):
    grid = (n_rows // tile,)
    in_specs = (
        [pl.BlockSpec((tile, a.shape[1]), lambda i: (i, 0)) for a in data]
        + [pl.BlockSpec(w.shape, lambda i: (0,) * w.ndim) for w in weights]
    )
    out_specs = [pl.BlockSpec((tile, d), lambda i: (i, 0)) for d in out_dims]
    out_shape = [jax.ShapeDtypeStruct((n_rows, d), jnp.float32) for d in out_dims]
    out = pl.pallas_call(
        body, grid=grid, in_specs=in_specs, out_specs=out_specs, out_shape=out_shape
    )(*data, *weights)
    return out


def _node_enc_body(na, W0, b0, W1, b1, Ws, Wd, x_o, ps_o, pd_o):
    h = _relu(_mm(na[...], W0[...]) + b0[...])
    x = _mm(h, W1[...]) + b1[...]
    x_o[...] = x
    ps_o[...] = _mm(x, Ws[...])
    pd_o[...] = _mm(x, Wd[...])


def _edge1_body(ea, g, We0, be0, We1, be1, Wm0e, bm0, Wm1, bm1,
                Wu0, bu0, Wu1, bu1, m_o, e_o):
    h0 = _relu(_mm(ea[...], We0[...]) + be0[...])
    e = _mm(h0, We1[...]) + be1[...]
    h = _relu(_mm(e, Wm0e[...]) + g[...] + bm0[...])
    m = _mm(h, Wm1[...]) + bm1[...]
    hu = _relu(_mm(m, Wu0[...]) + bu0[...])
    m_o[...] = m
    e_o[...] = _mm(hu, Wu1[...]) + bu1[...] + e


def _edge_mp_body(e_in, g, Wm0e, bm0, Wm1, bm1, Wu0, bu0, Wu1, bu1, m_o, e_o):
    e = e_in[...]
    h = _relu(_mm(e, Wm0e[...]) + g[...] + bm0[...])
    m = _mm(h, Wm1[...]) + bm1[...]
    hu = _relu(_mm(m, Wu0[...]) + bu0[...])
    m_o[...] = m
    e_o[...] = _mm(hu, Wu1[...]) + bu1[...] + e


def _node_mp_body(x_in, a0, a1, a2, a3, Wnx, Wna, bn0, Wn1, bn1, Ws, Wd,
                  x_o, ps_o, pd_o):
    x = x_in[...]
    aggr = (a0[...] + a1[...]) + (a2[...] + a3[...])
    h = _relu(_mm(x, Wnx[...]) + _mm(aggr, Wna[...]) + bn0[...])
    x2 = _mm(h, Wn1[...]) + bn1[...] + x
    x_o[...] = x2
    ps_o[...] = _mm(x2, Ws[...])
    pd_o[...] = _mm(x2, Wd[...])


def _node_mp_dec_body(x_in, a0, a1, a2, a3, Wnx, Wna, bn0, Wn1, bn1,
                      Wc0, bc0, Wc1, bc1, Ws, Wd, no_o, ps_o, pd_o):
    x = x_in[...]
    aggr = (a0[...] + a1[...]) + (a2[...] + a3[...])
    h = _relu(_mm(x, Wnx[...]) + _mm(aggr, Wna[...]) + bn0[...])
    x2 = _mm(h, Wn1[...]) + bn1[...] + x
    hd = _relu(_mm(x2, Wc0[...]) + bc0[...])
    no_o[...] = _mm(hd, Wc1[...]) + bc1[...]
    ps_o[...] = _mm(x2, Ws[...])
    pd_o[...] = _mm(x2, Wd[...])


def _edge_dec_body(e_in, g, Wd0e, bd0, Wd1, bd1, o):
    h = _relu(_mm(e_in[...], Wd0e[...]) + g[...] + bd0[...])
    o[...] = _mm(h, Wd1[...]) + bd1[...]


def _sc_mesh():
    return plsc.VectorSubcoreMesh(core_axis_name="c", subcore_axis_name="s")


def _sc_params():
    cp = pltpu.CompilerParams()
    if "needs_layout_passes" in pltpu.CompilerParams.__dataclass_fields__:
        cp = dataclasses.replace(cp, needs_layout_passes=False)
    return cp


def _chunking(n_edges):
    epw = n_edges // SC_WORKERS
    ch = 80 if epw % 80 == 0 else 40
    n_chunks = epw // ch
    assert epw % ch == 0 and n_chunks % 2 == 1
    return epw, ch, n_chunks


def _sc_gather_add(ps, pd, src, dst):
    n_edges = src.shape[0]
    epw, ch, n_chunks = _chunking(n_edges)

    def body(ps_hbm, pd_hbm, src_hbm, dst_hbm, g_hbm,
             ixs, ixd, bs0, bd0, bo0, bs1, bd1, bo1, sg0, sg1, ss0, ss1):
        wid = lax.axis_index("c") * SC_TILES + lax.axis_index("s")
        tb = wid * epw
        pltpu.sync_copy(src_hbm.at[pl.ds(tb, epw)], ixs)
        pltpu.sync_copy(dst_hbm.at[pl.ds(tb, epw)], ixd)

        def start(kk, bs, bd, sg):
            off = kk * ch
            c1 = pltpu.async_copy(ps_hbm.at[ixs.at[pl.ds(off, ch)]], bs, sg)
            c2 = pltpu.async_copy(pd_hbm.at[ixd.at[pl.ds(off, ch)]], bd, sg)
            return c1, c2

        def finish(kk, cps, bs, bd, bo, ss, guard):
            base = tb + kk * ch

            @pl.when(guard)
            def _wait_prev_store():
                pltpu.make_async_copy(bo, g_hbm.at[pl.ds(base, ch)], ss).wait()

            cps[0].wait()
            cps[1].wait()

            @pl.loop(0, ch)
            def _row(r):
                for cc in range(0, F, 16):
                    bo[r, pl.ds(cc, 16)] = (
                        bs[r, pl.ds(cc, 16)] + bd[r, pl.ds(cc, 16)]
                    )

            pltpu.async_copy(bo, g_hbm.at[pl.ds(base, ch)], ss)

        @pl.loop(0, n_chunks - 1, step=2)
        def _pair(k):
            ca = start(k, bs0, bd0, sg0)
            cb = start(k + 1, bs1, bd1, sg1)
            finish(k, ca, bs0, bd0, bo0, ss0, k > 0)
            finish(k + 1, cb, bs1, bd1, bo1, ss1, k > 0)

        klast = n_chunks - 1
        ca = start(klast, bs0, bd0, sg0)
        finish(klast, ca, bs0, bd0, bo0, ss0, True)
        pltpu.make_async_copy(
            bo0, g_hbm.at[pl.ds(tb + klast * ch, ch)], ss0).wait()
        pltpu.make_async_copy(
            bo1, g_hbm.at[pl.ds(tb + (klast - 1) * ch, ch)], ss1).wait()

    f = pl.kernel(
        body,
        out_type=jax.ShapeDtypeStruct((n_edges, F), jnp.float32),
        mesh=_sc_mesh(),
        compiler_params=_sc_params(),
        scratch_types=[
            pltpu.VMEM((epw,), jnp.int32),
            pltpu.VMEM((epw,), jnp.int32),
            pltpu.VMEM((ch, F), jnp.float32),
            pltpu.VMEM((ch, F), jnp.float32),
            pltpu.VMEM((ch, F), jnp.float32),
            pltpu.VMEM((ch, F), jnp.float32),
            pltpu.VMEM((ch, F), jnp.float32),
            pltpu.VMEM((ch, F), jnp.float32),
            pltpu.SemaphoreType.DMA,
            pltpu.SemaphoreType.DMA,
            pltpu.SemaphoreType.DMA,
            pltpu.SemaphoreType.DMA,
        ],
    )
    return f(ps, pd, src, dst)


def _sc_segment_sum(m, dst):
    n_edges = m.shape[0]
    epw, ch, n_chunks = _chunking(n_edges)

    def body(m_hbm, dst_hbm, out_hbm, mb0, mb1, ix0, ix1, shared, sm0, sm1):
        c = lax.axis_index("c")
        s = lax.axis_index("s")
        wid = c * SC_TILES + s

        @pl.loop(0, ch)
        def _zrow(r):
            @pl.loop(0, F, step=16)
            def _zcol(cc):
                mb0[r, pl.ds(cc, 16)] = jnp.zeros((16,), jnp.float32)

        @pl.loop(0, ROWS_PER_TILE, step=ch)
        def _zcopy(j):
            pltpu.sync_copy(mb0, shared.at[pl.ds(s * ROWS_PER_TILE + j, ch)])

        plsc.subcore_barrier()

        tb = wid * epw

        @pl.loop(0, n_chunks - 1, step=2)
        def _pair(k):
            ca = pltpu.async_copy(m_hbm.at[pl.ds(tb + k * ch, ch)], mb0, sm0)
            cb = pltpu.async_copy(
                m_hbm.at[pl.ds(tb + (k + 1) * ch, ch)], mb1, sm1)
            pltpu.sync_copy(dst_hbm.at[pl.ds(tb + k * ch, ch)], ix0)
            pltpu.sync_copy(dst_hbm.at[pl.ds(tb + (k + 1) * ch, ch)], ix1)
            ca.wait()
            pltpu.sync_copy(mb0, shared.at[ix0], add=True)
            cb.wait()
            pltpu.sync_copy(mb1, shared.at[ix1], add=True)

        klast = n_chunks - 1
        cl = pltpu.async_copy(m_hbm.at[pl.ds(tb + klast * ch, ch)], mb0, sm0)
        pltpu.sync_copy(dst_hbm.at[pl.ds(tb + klast * ch, ch)], ix0)
        cl.wait()
        pltpu.sync_copy(mb0, shared.at[ix0], add=True)

        plsc.subcore_barrier()
        out_base = c * N_PAD + s * ROWS_PER_TILE
        pltpu.sync_copy(
            shared.at[pl.ds(s * ROWS_PER_TILE, ROWS_PER_TILE)],
            out_hbm.at[pl.ds(out_base, ROWS_PER_TILE)],
        )

    f = pl.kernel(
        body,
        out_type=jax.ShapeDtypeStruct((2 * N_PAD, F), jnp.float32),
        mesh=_sc_mesh(),
        scratch_types=[
            pltpu.VMEM((ch, F), jnp.float32),
            pltpu.VMEM((ch, F), jnp.float32),
            pltpu.VMEM((ch,), jnp.int32),
            pltpu.VMEM((ch,), jnp.int32),
            pltpu.VMEM_SHARED((N_PAD, F), jnp.float32),
            pltpu.SemaphoreType.DMA,
            pltpu.SemaphoreType.DMA,
        ],
    )
    return f(m, dst)


def kernel(node_attr, edge_index, edge_attr, params):
    eh = N_EDGES // N_HALF
    srcs = [edge_index[0, i * eh:(i + 1) * eh] for i in range(N_HALF)]
    dsts = [edge_index[1, i * eh:(i + 1) * eh] for i in range(N_HALF)]
    eas = [edge_attr[i * eh:(i + 1) * eh] for i in range(N_HALF)]

    (Wne0, bne0), (Wne1, bne1) = params["node_enc"]
    (Wee0, bee0), (Wee1, bee1) = params["edge_enc"]
    (Wnd0, bnd0), (Wnd1, bnd1) = params["node_dec"]
    (Wd0, bd0), (Wd1, bd1) = params["edge_dec"]
    mp = params["mp"]

    def msg_split(lp):
        (Wm0, bm0), (Wm1, bm1) = lp["msg"]
        return Wm0[:F], Wm0[F:2 * F], Wm0[2 * F:], bm0, Wm1, bm1

    def node_split(lp):
        (Wn0, bn0), (Wn1, bn1) = lp["node"]
        return Wn0[:F], Wn0[F:], bn0, Wn1, bn1

    r = lambda b: b.reshape(1, -1)
    w = lambda W: W.astype(jnp.bfloat16)

    Wm0e_1, Wm0s_1, Wm0d_1, bm0_1, Wm1_1, bm1_1 = msg_split(mp[0])
    Wm0e_2, Wm0s_2, Wm0d_2, bm0_2, Wm1_2, bm1_2 = msg_split(mp[1])
    (Wu0_1, bu0_1), (Wu1_1, bu1_1) = mp[0]["edge"]
    (Wu0_2, bu0_2), (Wu1_2, bu1_2) = mp[1]["edge"]
    Wnx_1, Wna_1, bn0_1, Wn1_1, bn1_1 = node_split(mp[0])
    Wnx_2, Wna_2, bn0_2, Wn1_2, bn1_2 = node_split(mp[1])
    Wd0e, Wd0s, Wd0d = Wd0[:F], Wd0[F:2 * F], Wd0[2 * F:]

    x0, p1s, p1d = _rowcall(
        _node_enc_body, N_NODES, N_TILE, [node_attr],
        [w(Wne0), r(bne0), w(Wne1), r(bne1), w(Wm0s_1), w(Wm0d_1)], [F, F, F])

    g1 = [_sc_gather_add(p1s, p1d, srcs[i], dsts[i]) for i in range(N_HALF)]
    me1 = [_rowcall(
        _edge1_body, eh, E_TILE, [eas[i], g1[i]],
        [w(Wee0), r(bee0), w(Wee1), r(bee1), w(Wm0e_1), r(bm0_1), w(Wm1_1),
         r(bm1_1), w(Wu0_1), r(bu0_1), w(Wu1_1), r(bu1_1)], [F, F]) for i in range(N_HALF)]
    parts1 = [_sc_segment_sum(me1[i][0], dsts[i]) for i in range(N_HALF)]
    x1, p2s, p2d = _rowcall(
        _node_mp_body, N_NODES, N_TILE,
        [x0,
         parts1[0][:N_NODES], parts1[0][N_PAD:N_PAD + N_NODES],
         parts1[1][:N_NODES], parts1[1][N_PAD:N_PAD + N_NODES]],
        [w(Wnx_1), w(Wna_1), r(bn0_1), w(Wn1_1), r(bn1_1), w(Wm0s_2), w(Wm0d_2)], [F, F, F])

    g2 = [_sc_gather_add(p2s, p2d, srcs[i], dsts[i]) for i in range(N_HALF)]
    me2 = [_rowcall(
        _edge_mp_body, eh, E_TILE, [me1[i][1], g2[i]],
        [w(Wm0e_2), r(bm0_2), w(Wm1_2), r(bm1_2),
         w(Wu0_2), r(bu0_2), w(Wu1_2), r(bu1_2)], [F, F]) for i in range(N_HALF)]
    parts2 = [_sc_segment_sum(me2[i][0], dsts[i]) for i in range(N_HALF)]
    node_out, pds, pdd = _rowcall(
        _node_mp_dec_body, N_NODES, N_TILE,
        [x1,
         parts2[0][:N_NODES], parts2[0][N_PAD:N_PAD + N_NODES],
         parts2[1][:N_NODES], parts2[1][N_PAD:N_PAD + N_NODES]],
        [w(Wnx_2), w(Wna_2), r(bn0_2), w(Wn1_2), r(bn1_2),
         w(Wnd0), r(bnd0), w(Wnd1), r(bnd1), w(Wd0s), w(Wd0d)], [F, F, F])

    g3 = [_sc_gather_add(pds, pdd, srcs[i], dsts[i]) for i in range(N_HALF)]
    eo = [_rowcall(
        _edge_dec_body, eh, E_TILE, [me2[i][1], g3[i]],
        [w(Wd0e), r(bd0), w(Wd1), r(bd1)], [16])[0] for i in range(N_HALF)]
    edge_out = jnp.concatenate(eo, axis=0)

    return (node_out, edge_out)


# speedup vs baseline: 1.2795x; 1.0358x over previous
import dataclasses

import jax
import jax.numpy as jnp
from jax import lax
from jax.experimental import pallas as pl
from jax.experimental.pallas import tpu as pltpu
from jax.experimental.pallas import tpu_sc as plsc

N_NODES = 10000
N_EDGES = 320000
F = 128

E_TILE = 4000
N_TILE = 1000
N_HALF = 2

SC_TILES = 16
SC_WORKERS = 2 * SC_TILES
N_PAD = 10240
ROWS_PER_TILE = N_PAD // SC_TILES


FH = F // 2


def _relu(v):
    return jnp.maximum(v, 0.0)


def _mm(a, b):
    return jnp.dot(a, b, preferred_element_type=jnp.float32)


def _pack_bf16(p):
    u = lax.bitcast_convert_type(
        p[:, :FH].astype(jnp.bfloat16), jnp.uint16).astype(jnp.uint32)
    v = lax.bitcast_convert_type(
        p[:, FH:].astype(jnp.bfloat16), jnp.uint16).astype(jnp.uint32)
    return lax.bitcast_convert_type(u | (v << 16), jnp.float32)


def _unpack_bf16(gp):
    w = lax.bitcast_convert_type(gp, jnp.uint32)
    lo = lax.bitcast_convert_type((w & 0xFFFF).astype(jnp.uint16), jnp.bfloat16)
    hi = lax.bitcast_convert_type((w >> 16).astype(jnp.uint16), jnp.bfloat16)
    return jnp.concatenate(
        [lo.astype(jnp.float32), hi.astype(jnp.float32)], axis=-1)


def _rowcall(body, n_rows, tile, data, weights, out_dims):
    grid = (n_rows // tile,)
    in_specs = (
        [pl.BlockSpec((tile, a.shape[1]), lambda i: (i, 0)) for a in data]
        + [pl.BlockSpec(w.shape, lambda i: (0,) * w.ndim) for w in weights]
    )
    out_specs = [pl.BlockSpec((tile, d), lambda i: (i, 0)) for d in out_dims]
    out_shape = [jax.ShapeDtypeStruct((n_rows, d), jnp.float32) for d in out_dims]
    out = pl.pallas_call(
        body, grid=grid, in_specs=in_specs, out_specs=out_specs, out_shape=out_shape
    )(*data, *weights)
    return out


def _node_enc_body(na, W0, b0, W1, b1, Ws, Wd, x_o, ps_o, pd_o):
    h = _relu(_mm(na[...], W0[...]) + b0[...])
    x = _mm(h, W1[...]) + b1[...]
    x_o[...] = x
    ps_o[...] = _mm(x, Ws[...])
    pd_o[...] = _mm(x, Wd[...])


def _edge1_body(ea, g, We0, be0, We1, be1, Wm0e, bm0, Wm1, bm1,
                Wu0, bu0, Wu1, bu1, m_o, e_o):
    h0 = _relu(_mm(ea[...], We0[...]) + be0[...])
    e = _mm(h0, We1[...]) + be1[...]
    h = _relu(_mm(e, Wm0e[...]) + g[...] + bm0[...])
    m = _mm(h, Wm1[...]) + bm1[...]
    hu = _relu(_mm(m, Wu0[...]) + bu0[...])
    m_o[...] = m
    e_o[...] = _mm(hu, Wu1[...]) + bu1[...] + e


def _edge_mp_body(e_in, g, Wm0e, bm0, Wm1, bm1, Wu0, bu0, Wu1, bu1, m_o, e_o):
    e = e_in[...]
    h = _relu(_mm(e, Wm0e[...]) + g[...] + bm0[...])
    m = _mm(h, Wm1[...]) + bm1[...]
    hu = _relu(_mm(m, Wu0[...]) + bu0[...])
    m_o[...] = m
    e_o[...] = _mm(hu, Wu1[...]) + bu1[...] + e


def _node_mp_body(x_in, a0, a1, a2, a3, Wnx, Wna, bn0, Wn1, bn1, Ws, Wd,
                  x_o, ps_o, pd_o):
    x = x_in[...]
    aggr = (a0[...] + a1[...]) + (a2[...] + a3[...])
    h = _relu(_mm(x, Wnx[...]) + _mm(aggr, Wna[...]) + bn0[...])
    x2 = _mm(h, Wn1[...]) + bn1[...] + x
    x_o[...] = x2
    ps_o[...] = _mm(x2, Ws[...])
    pd_o[...] = _mm(x2, Wd[...])


def _node_mp_dec_body(x_in, a0, a1, a2, a3, Wnx, Wna, bn0, Wn1, bn1,
                      Wc0, bc0, Wc1, bc1, Ws, Wd, no_o, ps_o, pd_o):
    x = x_in[...]
    aggr = (a0[...] + a1[...]) + (a2[...] + a3[...])
    h = _relu(_mm(x, Wnx[...]) + _mm(aggr, Wna[...]) + bn0[...])
    x2 = _mm(h, Wn1[...]) + bn1[...] + x
    hd = _relu(_mm(x2, Wc0[...]) + bc0[...])
    no_o[...] = _mm(hd, Wc1[...]) + bc1[...]
    ps_o[...] = _mm(x2, Ws[...])
    pd_o[...] = _mm(x2, Wd[...])


def _edge_dec_body(e_in, g, Wd0e, bd0, Wd1, bd1, o):
    h = _relu(_mm(e_in[...], Wd0e[...]) + g[...] + bd0[...])
    o[...] = _mm(h, Wd1[...]) + bd1[...]


def _sc_mesh():
    return plsc.VectorSubcoreMesh(core_axis_name="c", subcore_axis_name="s")


def _sc_params():
    cp = pltpu.CompilerParams()
    if "needs_layout_passes" in pltpu.CompilerParams.__dataclass_fields__:
        cp = dataclasses.replace(cp, needs_layout_passes=False)
    return cp


def _chunking(n_edges):
    epw = n_edges // SC_WORKERS
    ch = 80 if epw % 80 == 0 else 40
    n_chunks = epw // ch
    assert epw % ch == 0 and n_chunks % 2 == 1
    return epw, ch, n_chunks


def _sc_gather_add(ps, pd, src, dst):
    n_edges = src.shape[0]
    epw, ch, n_chunks = _chunking(n_edges)

    def body(ps_hbm, pd_hbm, src_hbm, dst_hbm, g_hbm,
             ixs, ixd, bs0, bd0, bo0, bs1, bd1, bo1, sg0, sg1, ss0, ss1):
        wid = lax.axis_index("c") * SC_TILES + lax.axis_index("s")
        tb = wid * epw
        pltpu.sync_copy(src_hbm.at[pl.ds(tb, epw)], ixs)
        pltpu.sync_copy(dst_hbm.at[pl.ds(tb, epw)], ixd)

        def start(kk, bs, bd, sg):
            off = kk * ch
            c1 = pltpu.async_copy(ps_hbm.at[ixs.at[pl.ds(off, ch)]], bs, sg)
            c2 = pltpu.async_copy(pd_hbm.at[ixd.at[pl.ds(off, ch)]], bd, sg)
            return c1, c2

        def finish(kk, cps, bs, bd, bo, ss, guard):
            base = tb + kk * ch

            @pl.when(guard)
            def _wait_prev_store():
                pltpu.make_async_copy(bo, g_hbm.at[pl.ds(base, ch)], ss).wait()

            cps[0].wait()
            cps[1].wait()

            @pl.loop(0, ch)
            def _row(r):
                for cc in range(0, F, 16):
                    bo[r, pl.ds(cc, 16)] = (
                        bs[r, pl.ds(cc, 16)] + bd[r, pl.ds(cc, 16)]
                    )

            pltpu.async_copy(bo, g_hbm.at[pl.ds(base, ch)], ss)

        @pl.loop(0, n_chunks - 1, step=2)
        def _pair(k):
            ca = start(k, bs0, bd0, sg0)
            cb = start(k + 1, bs1, bd1, sg1)
            finish(k, ca, bs0, bd0, bo0, ss0, k > 0)
            finish(k + 1, cb, bs1, bd1, bo1, ss1, k > 0)

        klast = n_chunks - 1
        ca = start(klast, bs0, bd0, sg0)
        finish(klast, ca, bs0, bd0, bo0, ss0, True)
        pltpu.make_async_copy(
            bo0, g_hbm.at[pl.ds(tb + klast * ch, ch)], ss0).wait()
        pltpu.make_async_copy(
            bo1, g_hbm.at[pl.ds(tb + (klast - 1) * ch, ch)], ss1).wait()

    f = pl.kernel(
        body,
        out_type=jax.ShapeDtypeStruct((n_edges, F), jnp.float32),
        mesh=_sc_mesh(),
        compiler_params=_sc_params(),
        scratch_types=[
            pltpu.VMEM((epw,), jnp.int32),
            pltpu.VMEM((epw,), jnp.int32),
            pltpu.VMEM((ch, F), jnp.float32),
            pltpu.VMEM((ch, F), jnp.float32),
            pltpu.VMEM((ch, F), jnp.float32),
            pltpu.VMEM((ch, F), jnp.float32),
            pltpu.VMEM((ch, F), jnp.float32),
            pltpu.VMEM((ch, F), jnp.float32),
            pltpu.SemaphoreType.DMA,
            pltpu.SemaphoreType.DMA,
            pltpu.SemaphoreType.DMA,
            pltpu.SemaphoreType.DMA,
        ],
    )
    return f(ps, pd, src, dst)


def _sc_segment_sum(m, dst):
    n_edges = m.shape[0]
    epw, ch, n_chunks = _chunking(n_edges)

    def body(m_hbm, dst_hbm, out_hbm, mb0, mb1, ix0, ix1, shared, sm0, sm1):
        c = lax.axis_index("c")
        s = lax.axis_index("s")
        wid = c * SC_TILES + s

        @pl.loop(0, ch)
        def _zrow(r):
            @pl.loop(0, F, step=16)
            def _zcol(cc):
                mb0[r, pl.ds(cc, 16)] = jnp.zeros((16,), jnp.float32)

        @pl.loop(0, ROWS_PER_TILE, step=ch)
        def _zcopy(j):
            pltpu.sync_copy(mb0, shared.at[pl.ds(s * ROWS_PER_TILE + j, ch)])

        plsc.subcore_barrier()

        tb = wid * epw

        @pl.loop(0, n_chunks - 1, step=2)
        def _pair(k):
            ca = pltpu.async_copy(m_hbm.at[pl.ds(tb + k * ch, ch)], mb0, sm0)
            cb = pltpu.async_copy(
                m_hbm.at[pl.ds(tb + (k + 1) * ch, ch)], mb1, sm1)
            pltpu.sync_copy(dst_hbm.at[pl.ds(tb + k * ch, ch)], ix0)
            pltpu.sync_copy(dst_hbm.at[pl.ds(tb + (k + 1) * ch, ch)], ix1)
            ca.wait()
            pltpu.sync_copy(mb0, shared.at[ix0], add=True)
            cb.wait()
            pltpu.sync_copy(mb1, shared.at[ix1], add=True)

        klast = n_chunks - 1
        cl = pltpu.async_copy(m_hbm.at[pl.ds(tb + klast * ch, ch)], mb0, sm0)
        pltpu.sync_copy(dst_hbm.at[pl.ds(tb + klast * ch, ch)], ix0)
        cl.wait()
        pltpu.sync_copy(mb0, shared.at[ix0], add=True)

        plsc.subcore_barrier()
        out_base = c * N_PAD + s * ROWS_PER_TILE
        pltpu.sync_copy(
            shared.at[pl.ds(s * ROWS_PER_TILE, ROWS_PER_TILE)],
            out_hbm.at[pl.ds(out_base, ROWS_PER_TILE)],
        )

    f = pl.kernel(
        body,
        out_type=jax.ShapeDtypeStruct((2 * N_PAD, F), jnp.float32),
        mesh=_sc_mesh(),
        scratch_types=[
            pltpu.VMEM((ch, F), jnp.float32),
            pltpu.VMEM((ch, F), jnp.float32),
            pltpu.VMEM((ch,), jnp.int32),
            pltpu.VMEM((ch,), jnp.int32),
            pltpu.VMEM_SHARED((N_PAD, F), jnp.float32),
            pltpu.SemaphoreType.DMA,
            pltpu.SemaphoreType.DMA,
        ],
    )
    return f(m, dst)


def kernel(node_attr, edge_index, edge_attr, params):
    eh = N_EDGES // N_HALF
    srcs = [edge_index[0, i * eh:(i + 1) * eh] for i in range(N_HALF)]
    dsts = [edge_index[1, i * eh:(i + 1) * eh] for i in range(N_HALF)]
    eas = [edge_attr[i * eh:(i + 1) * eh] for i in range(N_HALF)]

    (Wne0, bne0), (Wne1, bne1) = params["node_enc"]
    (Wee0, bee0), (Wee1, bee1) = params["edge_enc"]
    (Wnd0, bnd0), (Wnd1, bnd1) = params["node_dec"]
    (Wd0, bd0), (Wd1, bd1) = params["edge_dec"]
    mp = params["mp"]

    def msg_split(lp):
        (Wm0, bm0), (Wm1, bm1) = lp["msg"]
        return Wm0[:F], Wm0[F:2 * F], Wm0[2 * F:], bm0, Wm1, bm1

    def node_split(lp):
        (Wn0, bn0), (Wn1, bn1) = lp["node"]
        return Wn0[:F], Wn0[F:], bn0, Wn1, bn1

    r = lambda b: b.reshape(1, -1)
    w = lambda W: W

    Wm0e_1, Wm0s_1, Wm0d_1, bm0_1, Wm1_1, bm1_1 = msg_split(mp[0])
    Wm0e_2, Wm0s_2, Wm0d_2, bm0_2, Wm1_2, bm1_2 = msg_split(mp[1])
    (Wu0_1, bu0_1), (Wu1_1, bu1_1) = mp[0]["edge"]
    (Wu0_2, bu0_2), (Wu1_2, bu1_2) = mp[1]["edge"]
    Wnx_1, Wna_1, bn0_1, Wn1_1, bn1_1 = node_split(mp[0])
    Wnx_2, Wna_2, bn0_2, Wn1_2, bn1_2 = node_split(mp[1])
    Wd0e, Wd0s, Wd0d = Wd0[:F], Wd0[F:2 * F], Wd0[2 * F:]

    x0, p1s, p1d = _rowcall(
        _node_enc_body, N_NODES, N_TILE, [node_attr],
        [w(Wne0), r(bne0), w(Wne1), r(bne1), w(Wm0s_1), w(Wm0d_1)], [F, F, F])

    g1 = [_sc_gather_add(p1s, p1d, srcs[i], dsts[i]) for i in range(N_HALF)]
    me1 = [_rowcall(
        _edge1_body, eh, E_TILE, [eas[i], g1[i]],
        [w(Wee0), r(bee0), w(Wee1), r(bee1), w(Wm0e_1), r(bm0_1), w(Wm1_1),
         r(bm1_1), w(Wu0_1), r(bu0_1), w(Wu1_1), r(bu1_1)], [F, F]) for i in range(N_HALF)]
    parts1 = [_sc_segment_sum(me1[i][0], dsts[i]) for i in range(N_HALF)]
    x1, p2s, p2d = _rowcall(
        _node_mp_body, N_NODES, N_TILE,
        [x0,
         parts1[0][:N_NODES], parts1[0][N_PAD:N_PAD + N_NODES],
         parts1[1][:N_NODES], parts1[1][N_PAD:N_PAD + N_NODES]],
        [w(Wnx_1), w(Wna_1), r(bn0_1), w(Wn1_1), r(bn1_1), w(Wm0s_2), w(Wm0d_2)], [F, F, F])

    g2 = [_sc_gather_add(p2s, p2d, srcs[i], dsts[i]) for i in range(N_HALF)]
    me2 = [_rowcall(
        _edge_mp_body, eh, E_TILE, [me1[i][1], g2[i]],
        [w(Wm0e_2), r(bm0_2), w(Wm1_2), r(bm1_2),
         w(Wu0_2), r(bu0_2), w(Wu1_2), r(bu1_2)], [F, F]) for i in range(N_HALF)]
    parts2 = [_sc_segment_sum(me2[i][0], dsts[i]) for i in range(N_HALF)]
    node_out, pds, pdd = _rowcall(
        _node_mp_dec_body, N_NODES, N_TILE,
        [x1,
         parts2[0][:N_NODES], parts2[0][N_PAD:N_PAD + N_NODES],
         parts2[1][:N_NODES], parts2[1][N_PAD:N_PAD + N_NODES]],
        [w(Wnx_2), w(Wna_2), r(bn0_2), w(Wn1_2), r(bn1_2),
         w(Wnd0), r(bnd0), w(Wnd1), r(bnd1), w(Wd0s), w(Wd0d)], [F, F, F])

    g3 = [_sc_gather_add(pds, pdd, srcs[i], dsts[i]) for i in range(N_HALF)]
    eo = [_rowcall(
        _edge_dec_body, eh, E_TILE, [me2[i][1], g3[i]],
        [w(Wd0e), r(bd0), w(Wd1), r(bd1)], [16])[0] for i in range(N_HALF)]
    edge_out = jnp.concatenate(eo, axis=0)

    return (node_out, edge_out)


# speedup vs baseline: 1.2910x; 1.0090x over previous
import dataclasses

import jax
import jax.numpy as jnp
from jax import lax
from jax.experimental import pallas as pl
from jax.experimental.pallas import tpu as pltpu
from jax.experimental.pallas import tpu_sc as plsc

N_NODES = 10000
N_EDGES = 320000
F = 128

E_TILE = 5000
N_TILE = 2000
N_HALF = 2

SC_TILES = 16
SC_WORKERS = 2 * SC_TILES
N_PAD = 10240
ROWS_PER_TILE = N_PAD // SC_TILES


FH = F // 2


def _relu(v):
    return jnp.maximum(v, 0.0)


def _mm(a, b):
    return jnp.dot(a, b, preferred_element_type=jnp.float32)


def _pack_bf16(p):
    u = lax.bitcast_convert_type(
        p[:, :FH].astype(jnp.bfloat16), jnp.uint16).astype(jnp.uint32)
    v = lax.bitcast_convert_type(
        p[:, FH:].astype(jnp.bfloat16), jnp.uint16).astype(jnp.uint32)
    return lax.bitcast_convert_type(u | (v << 16), jnp.float32)


def _unpack_bf16(gp):
    w = lax.bitcast_convert_type(gp, jnp.uint32)
    lo = lax.bitcast_convert_type((w & 0xFFFF).astype(jnp.uint16), jnp.bfloat16)
    hi = lax.bitcast_convert_type((w >> 16).astype(jnp.uint16), jnp.bfloat16)
    return jnp.concatenate(
        [lo.astype(jnp.float32), hi.astype(jnp.float32)], axis=-1)


def _rowcall(body, n_rows, tile, data, weights, out_dims):
    grid = (n_rows // tile,)
    in_specs = (
        [pl.BlockSpec((tile, a.shape[1]), lambda i: (i, 0)) for a in data]
        + [pl.BlockSpec(w.shape, lambda i: (0,) * w.ndim) for w in weights]
    )
    out_specs = [pl.BlockSpec((tile, d), lambda i: (i, 0)) for d in out_dims]
    out_shape = [jax.ShapeDtypeStruct((n_rows, d), jnp.float32) for d in out_dims]
    out = pl.pallas_call(
        body, grid=grid, in_specs=in_specs, out_specs=out_specs, out_shape=out_shape
    )(*data, *weights)
    return out


def _node_enc_body(na, W0, b0, W1, b1, Ws, Wd, x_o, ps_o, pd_o):
    h = _relu(_mm(na[...], W0[...]) + b0[...])
    x = _mm(h, W1[...]) + b1[...]
    x_o[...] = x
    ps_o[...] = _mm(x, Ws[...])
    pd_o[...] = _mm(x, Wd[...])


def _edge1_body(ea, g, We0, be0, We1, be1, Wm0e, bm0, Wm1, bm1,
                Wu0, bu0, Wu1, bu1, m_o, e_o):
    h0 = _relu(_mm(ea[...], We0[...]) + be0[...])
    e = _mm(h0, We1[...]) + be1[...]
    h = _relu(_mm(e, Wm0e[...]) + g[...] + bm0[...])
    m = _mm(h, Wm1[...]) + bm1[...]
    hu = _relu(_mm(m, Wu0[...]) + bu0[...])
    m_o[...] = m
    e_o[...] = _mm(hu, Wu1[...]) + bu1[...] + e


def _edge_mp_body(e_in, g, Wm0e, bm0, Wm1, bm1, Wu0, bu0, Wu1, bu1, m_o, e_o):
    e = e_in[...]
    h = _relu(_mm(e, Wm0e[...]) + g[...] + bm0[...])
    m = _mm(h, Wm1[...]) + bm1[...]
    hu = _relu(_mm(m, Wu0[...]) + bu0[...])
    m_o[...] = m
    e_o[...] = _mm(hu, Wu1[...]) + bu1[...] + e


def _node_mp_body(x_in, a0, a1, a2, a3, Wnx, Wna, bn0, Wn1, bn1, Ws, Wd,
                  x_o, ps_o, pd_o):
    x = x_in[...]
    aggr = (a0[...] + a1[...]) + (a2[...] + a3[...])
    h = _relu(_mm(x, Wnx[...]) + _mm(aggr, Wna[...]) + bn0[...])
    x2 = _mm(h, Wn1[...]) + bn1[...] + x
    x_o[...] = x2
    ps_o[...] = _mm(x2, Ws[...])
    pd_o[...] = _mm(x2, Wd[...])


def _node_mp_dec_body(x_in, a0, a1, a2, a3, Wnx, Wna, bn0, Wn1, bn1,
                      Wc0, bc0, Wc1, bc1, Ws, Wd, no_o, ps_o, pd_o):
    x = x_in[...]
    aggr = (a0[...] + a1[...]) + (a2[...] + a3[...])
    h = _relu(_mm(x, Wnx[...]) + _mm(aggr, Wna[...]) + bn0[...])
    x2 = _mm(h, Wn1[...]) + bn1[...] + x
    hd = _relu(_mm(x2, Wc0[...]) + bc0[...])
    no_o[...] = _mm(hd, Wc1[...]) + bc1[...]
    ps_o[...] = _mm(x2, Ws[...])
    pd_o[...] = _mm(x2, Wd[...])


def _edge_dec_body(e_in, g, Wd0e, bd0, Wd1, bd1, o):
    h = _relu(_mm(e_in[...], Wd0e[...]) + g[...] + bd0[...])
    o[...] = _mm(h, Wd1[...]) + bd1[...]


def _sc_mesh():
    return plsc.VectorSubcoreMesh(core_axis_name="c", subcore_axis_name="s")


def _sc_params():
    cp = pltpu.CompilerParams()
    if "needs_layout_passes" in pltpu.CompilerParams.__dataclass_fields__:
        cp = dataclasses.replace(cp, needs_layout_passes=False)
    return cp


def _chunking(n_edges):
    epw = n_edges // SC_WORKERS
    ch = 80 if epw % 80 == 0 else 40
    n_chunks = epw // ch
    assert epw % ch == 0 and n_chunks % 2 == 1
    return epw, ch, n_chunks


def _sc_gather_add(ps, pd, src, dst):
    n_edges = src.shape[0]
    epw, ch, n_chunks = _chunking(n_edges)

    def body(ps_hbm, pd_hbm, src_hbm, dst_hbm, g_hbm,
             ixs, ixd, bs0, bd0, bo0, bs1, bd1, bo1, sg0, sg1, ss0, ss1):
        wid = lax.axis_index("c") * SC_TILES + lax.axis_index("s")
        tb = wid * epw
        pltpu.sync_copy(src_hbm.at[pl.ds(tb, epw)], ixs)
        pltpu.sync_copy(dst_hbm.at[pl.ds(tb, epw)], ixd)

        def start(kk, bs, bd, sg):
            off = kk * ch
            c1 = pltpu.async_copy(ps_hbm.at[ixs.at[pl.ds(off, ch)]], bs, sg)
            c2 = pltpu.async_copy(pd_hbm.at[ixd.at[pl.ds(off, ch)]], bd, sg)
            return c1, c2

        def finish(kk, cps, bs, bd, bo, ss, guard):
            base = tb + kk * ch

            @pl.when(guard)
            def _wait_prev_store():
                pltpu.make_async_copy(bo, g_hbm.at[pl.ds(base, ch)], ss).wait()

            cps[0].wait()
            cps[1].wait()

            @pl.loop(0, ch)
            def _row(r):
                for cc in range(0, F, 16):
                    bo[r, pl.ds(cc, 16)] = (
                        bs[r, pl.ds(cc, 16)] + bd[r, pl.ds(cc, 16)]
                    )

            pltpu.async_copy(bo, g_hbm.at[pl.ds(base, ch)], ss)

        @pl.loop(0, n_chunks - 1, step=2)
        def _pair(k):
            ca = start(k, bs0, bd0, sg0)
            cb = start(k + 1, bs1, bd1, sg1)
            finish(k, ca, bs0, bd0, bo0, ss0, k > 0)
            finish(k + 1, cb, bs1, bd1, bo1, ss1, k > 0)

        klast = n_chunks - 1
        ca = start(klast, bs0, bd0, sg0)
        finish(klast, ca, bs0, bd0, bo0, ss0, True)
        pltpu.make_async_copy(
            bo0, g_hbm.at[pl.ds(tb + klast * ch, ch)], ss0).wait()
        pltpu.make_async_copy(
            bo1, g_hbm.at[pl.ds(tb + (klast - 1) * ch, ch)], ss1).wait()

    f = pl.kernel(
        body,
        out_type=jax.ShapeDtypeStruct((n_edges, F), jnp.float32),
        mesh=_sc_mesh(),
        compiler_params=_sc_params(),
        scratch_types=[
            pltpu.VMEM((epw,), jnp.int32),
            pltpu.VMEM((epw,), jnp.int32),
            pltpu.VMEM((ch, F), jnp.float32),
            pltpu.VMEM((ch, F), jnp.float32),
            pltpu.VMEM((ch, F), jnp.float32),
            pltpu.VMEM((ch, F), jnp.float32),
            pltpu.VMEM((ch, F), jnp.float32),
            pltpu.VMEM((ch, F), jnp.float32),
            pltpu.SemaphoreType.DMA,
            pltpu.SemaphoreType.DMA,
            pltpu.SemaphoreType.DMA,
            pltpu.SemaphoreType.DMA,
        ],
    )
    return f(ps, pd, src, dst)


def _sc_segment_sum(m, dst):
    n_edges = m.shape[0]
    epw, ch, n_chunks = _chunking(n_edges)

    def body(m_hbm, dst_hbm, out_hbm, mb0, mb1, ix0, ix1, shared, sm0, sm1):
        c = lax.axis_index("c")
        s = lax.axis_index("s")
        wid = c * SC_TILES + s

        @pl.loop(0, ch)
        def _zrow(r):
            @pl.loop(0, F, step=16)
            def _zcol(cc):
                mb0[r, pl.ds(cc, 16)] = jnp.zeros((16,), jnp.float32)

        @pl.loop(0, ROWS_PER_TILE, step=ch)
        def _zcopy(j):
            pltpu.sync_copy(mb0, shared.at[pl.ds(s * ROWS_PER_TILE + j, ch)])

        plsc.subcore_barrier()

        tb = wid * epw

        @pl.loop(0, n_chunks - 1, step=2)
        def _pair(k):
            ca = pltpu.async_copy(m_hbm.at[pl.ds(tb + k * ch, ch)], mb0, sm0)
            cb = pltpu.async_copy(
                m_hbm.at[pl.ds(tb + (k + 1) * ch, ch)], mb1, sm1)
            pltpu.sync_copy(dst_hbm.at[pl.ds(tb + k * ch, ch)], ix0)
            pltpu.sync_copy(dst_hbm.at[pl.ds(tb + (k + 1) * ch, ch)], ix1)
            ca.wait()
            pltpu.sync_copy(mb0, shared.at[ix0], add=True)
            cb.wait()
            pltpu.sync_copy(mb1, shared.at[ix1], add=True)

        klast = n_chunks - 1
        cl = pltpu.async_copy(m_hbm.at[pl.ds(tb + klast * ch, ch)], mb0, sm0)
        pltpu.sync_copy(dst_hbm.at[pl.ds(tb + klast * ch, ch)], ix0)
        cl.wait()
        pltpu.sync_copy(mb0, shared.at[ix0], add=True)

        plsc.subcore_barrier()
        out_base = c * N_PAD + s * ROWS_PER_TILE
        pltpu.sync_copy(
            shared.at[pl.ds(s * ROWS_PER_TILE, ROWS_PER_TILE)],
            out_hbm.at[pl.ds(out_base, ROWS_PER_TILE)],
        )

    f = pl.kernel(
        body,
        out_type=jax.ShapeDtypeStruct((2 * N_PAD, F), jnp.float32),
        mesh=_sc_mesh(),
        scratch_types=[
            pltpu.VMEM((ch, F), jnp.float32),
            pltpu.VMEM((ch, F), jnp.float32),
            pltpu.VMEM((ch,), jnp.int32),
            pltpu.VMEM((ch,), jnp.int32),
            pltpu.VMEM_SHARED((N_PAD, F), jnp.float32),
            pltpu.SemaphoreType.DMA,
            pltpu.SemaphoreType.DMA,
        ],
    )
    return f(m, dst)


def kernel(node_attr, edge_index, edge_attr, params):
    eh = N_EDGES // N_HALF
    srcs = [edge_index[0, i * eh:(i + 1) * eh] for i in range(N_HALF)]
    dsts = [edge_index[1, i * eh:(i + 1) * eh] for i in range(N_HALF)]
    eas = [edge_attr[i * eh:(i + 1) * eh] for i in range(N_HALF)]

    (Wne0, bne0), (Wne1, bne1) = params["node_enc"]
    (Wee0, bee0), (Wee1, bee1) = params["edge_enc"]
    (Wnd0, bnd0), (Wnd1, bnd1) = params["node_dec"]
    (Wd0, bd0), (Wd1, bd1) = params["edge_dec"]
    mp = params["mp"]

    def msg_split(lp):
        (Wm0, bm0), (Wm1, bm1) = lp["msg"]
        return Wm0[:F], Wm0[F:2 * F], Wm0[2 * F:], bm0, Wm1, bm1

    def node_split(lp):
        (Wn0, bn0), (Wn1, bn1) = lp["node"]
        return Wn0[:F], Wn0[F:], bn0, Wn1, bn1

    r = lambda b: b.reshape(1, -1)
    w = lambda W: W

    Wm0e_1, Wm0s_1, Wm0d_1, bm0_1, Wm1_1, bm1_1 = msg_split(mp[0])
    Wm0e_2, Wm0s_2, Wm0d_2, bm0_2, Wm1_2, bm1_2 = msg_split(mp[1])
    (Wu0_1, bu0_1), (Wu1_1, bu1_1) = mp[0]["edge"]
    (Wu0_2, bu0_2), (Wu1_2, bu1_2) = mp[1]["edge"]
    Wnx_1, Wna_1, bn0_1, Wn1_1, bn1_1 = node_split(mp[0])
    Wnx_2, Wna_2, bn0_2, Wn1_2, bn1_2 = node_split(mp[1])
    Wd0e, Wd0s, Wd0d = Wd0[:F], Wd0[F:2 * F], Wd0[2 * F:]

    x0, p1s, p1d = _rowcall(
        _node_enc_body, N_NODES, N_TILE, [node_attr],
        [w(Wne0), r(bne0), w(Wne1), r(bne1), w(Wm0s_1), w(Wm0d_1)], [F, F, F])

    g1 = [_sc_gather_add(p1s, p1d, srcs[i], dsts[i]) for i in range(N_HALF)]
    me1 = [_rowcall(
        _edge1_body, eh, E_TILE, [eas[i], g1[i]],
        [w(Wee0), r(bee0), w(Wee1), r(bee1), w(Wm0e_1), r(bm0_1), w(Wm1_1),
         r(bm1_1), w(Wu0_1), r(bu0_1), w(Wu1_1), r(bu1_1)], [F, F]) for i in range(N_HALF)]
    parts1 = [_sc_segment_sum(me1[i][0], dsts[i]) for i in range(N_HALF)]
    x1, p2s, p2d = _rowcall(
        _node_mp_body, N_NODES, N_TILE,
        [x0,
         parts1[0][:N_NODES], parts1[0][N_PAD:N_PAD + N_NODES],
         parts1[1][:N_NODES], parts1[1][N_PAD:N_PAD + N_NODES]],
        [w(Wnx_1), w(Wna_1), r(bn0_1), w(Wn1_1), r(bn1_1), w(Wm0s_2), w(Wm0d_2)], [F, F, F])

    g2 = [_sc_gather_add(p2s, p2d, srcs[i], dsts[i]) for i in range(N_HALF)]
    me2 = [_rowcall(
        _edge_mp_body, eh, E_TILE, [me1[i][1], g2[i]],
        [w(Wm0e_2), r(bm0_2), w(Wm1_2), r(bm1_2),
         w(Wu0_2), r(bu0_2), w(Wu1_2), r(bu1_2)], [F, F]) for i in range(N_HALF)]
    parts2 = [_sc_segment_sum(me2[i][0], dsts[i]) for i in range(N_HALF)]
    node_out, pds, pdd = _rowcall(
        _node_mp_dec_body, N_NODES, N_TILE,
        [x1,
         parts2[0][:N_NODES], parts2[0][N_PAD:N_PAD + N_NODES],
         parts2[1][:N_NODES], parts2[1][N_PAD:N_PAD + N_NODES]],
        [w(Wnx_2), w(Wna_2), r(bn0_2), w(Wn1_2), r(bn1_2),
         w(Wnd0), r(bnd0), w(Wnd1), r(bnd1), w(Wd0s), w(Wd0d)], [F, F, F])

    g3 = [_sc_gather_add(pds, pdd, srcs[i], dsts[i]) for i in range(N_HALF)]
    eo = [_rowcall(
        _edge_dec_body, eh, E_TILE, [me2[i][1], g3[i]],
        [w(Wd0e), r(bd0), w(Wd1), r(bd1)], [16])[0] for i in range(N_HALF)]
    edge_out = jnp.concatenate(eo, axis=0)

    return (node_out, edge_out)
